# baseline (speedup 1.0000x reference)
import functools

import jax
import jax.numpy as jnp
from jax import lax
from jax.experimental import pallas as pl
from jax.experimental.pallas import tpu as pltpu
from jax.experimental.pallas import tpu_sc as plsc

PI = 3.141592653589793
GAMMA = 12.0
HIDDEN_DIM = 64
EMBEDDING_RANGE = (GAMMA + 2.0) / HIDDEN_DIM

C_AX = 2.0 * PI / EMBEDDING_RANGE
C_AR = 4.0 * PI / EMBEDDING_RANGE

NW = 32
CHUNK = 128


def _body(table_hbm, idx_hbm, ax_hbm, ar_hbm, idx_v, rows_v, ax_v, ar_v, sem):
    wid = lax.axis_index("s") * 2 + lax.axis_index("c")
    n_chunks = idx_hbm.shape[0] // NW
    rows_per_w = n_chunks * CHUNK

    pltpu.sync_copy(idx_hbm.at[pl.ds(wid * n_chunks, n_chunks)], idx_v)

    def chunk_step(g, carry):
        pltpu.async_copy(table_hbm.at[idx_v.at[g]], rows_v, sem).wait()

        def row_step(r, carry2):
            for j in range(4):
                x = rows_v[r, pl.ds(j * 16, 16)]
                t = jnp.exp(x * C_AX)
                ax_v[r, pl.ds(j * 16, 16)] = PI - (2.0 * PI) / (t + 1.0)
            for j in range(4):
                x = rows_v[r, pl.ds(64 + j * 16, 16)]
                u = jnp.exp(x * C_AR)
                ar_v[r, pl.ds(j * 16, 16)] = PI - PI / (u + 1.0)
            return carry2

        lax.fori_loop(0, CHUNK, row_step, 0, unroll=2)

        base = wid * rows_per_w + g * CHUNK
        pltpu.sync_copy(ax_v, ax_hbm.at[pl.ds(base, CHUNK)])
        pltpu.sync_copy(ar_v, ar_hbm.at[pl.ds(base, CHUNK)])
        return carry

    lax.fori_loop(0, n_chunks, chunk_step, 0)


def kernel(entity_embedding, indices):
    b, l = indices.shape
    n = b * l
    assert n % (NW * CHUNK) == 0
    idx2d = indices.reshape(n // CHUNK, CHUNK)
    n_chunks = idx2d.shape[0] // NW

    mesh = plsc.VectorSubcoreMesh(core_axis_name="c", subcore_axis_name="s")
    run = functools.partial(
        pl.kernel,
        out_type=[
            jax.ShapeDtypeStruct((n, HIDDEN_DIM), jnp.float32),
            jax.ShapeDtypeStruct((n, HIDDEN_DIM), jnp.float32),
        ],
        mesh=mesh,
        scratch_types=[
            pltpu.VMEM((n_chunks, CHUNK), jnp.int32),
            pltpu.VMEM((CHUNK, 2 * HIDDEN_DIM), jnp.float32),
            pltpu.VMEM((CHUNK, HIDDEN_DIM), jnp.float32),
            pltpu.VMEM((CHUNK, HIDDEN_DIM), jnp.float32),
            pltpu.SemaphoreType.DMA,
        ],
    )(_body)

    ax, ar = run(entity_embedding, idx2d)
    return ax.reshape(b, l, HIDDEN_DIM), ar.reshape(b, l, HIDDEN_DIM)


# speedup vs baseline: 2.7744x; 2.7744x over previous
import functools

import jax
import jax.numpy as jnp
from jax import lax
from jax.experimental import pallas as pl
from jax.experimental.pallas import tpu as pltpu
from jax.experimental.pallas import tpu_sc as plsc

PI = 3.141592653589793
GAMMA = 12.0
HIDDEN_DIM = 64
EMBEDDING_RANGE = (GAMMA + 2.0) / HIDDEN_DIM

C_AX = 2.0 * PI / EMBEDDING_RANGE
C_AR = 4.0 * PI / EMBEDDING_RANGE

NW = 32
CHUNK = 128


def _body(table_hbm, idx_hbm, ax_hbm, ar_hbm, idx_v, rows_v, ax_v, ar_v, sem):
    wid = lax.axis_index("s") * 2 + lax.axis_index("c")
    n_chunks = idx_hbm.shape[0] // NW
    rows_per_w = n_chunks * CHUNK

    pltpu.sync_copy(idx_hbm.at[pl.ds(wid * n_chunks, n_chunks)], idx_v)

    def chunk_step(g, carry):
        pltpu.async_copy(table_hbm.at[idx_v.at[g]], rows_v, sem).wait()

        @plsc.parallel_loop(0, CHUNK, step=1, unroll=4)
        def row_step(r):
            for j in range(4):
                x = rows_v[r, pl.ds(j * 16, 16)]
                t = jnp.exp(x * C_AX)
                ax_v[r, pl.ds(j * 16, 16)] = PI - (2.0 * PI) / (t + 1.0)
            for j in range(4):
                x = rows_v[r, pl.ds(64 + j * 16, 16)]
                u = jnp.exp(x * C_AR)
                ar_v[r, pl.ds(j * 16, 16)] = PI - PI / (u + 1.0)

        base = wid * rows_per_w + g * CHUNK
        pltpu.sync_copy(ax_v, ax_hbm.at[pl.ds(base, CHUNK)])
        pltpu.sync_copy(ar_v, ar_hbm.at[pl.ds(base, CHUNK)])
        return carry

    lax.fori_loop(0, n_chunks, chunk_step, 0)


def kernel(entity_embedding, indices):
    b, l = indices.shape
    n = b * l
    assert n % (NW * CHUNK) == 0
    idx2d = indices.reshape(n // CHUNK, CHUNK)
    n_chunks = idx2d.shape[0] // NW

    mesh = plsc.VectorSubcoreMesh(core_axis_name="c", subcore_axis_name="s")
    run = functools.partial(
        pl.kernel,
        out_type=[
            jax.ShapeDtypeStruct((n, HIDDEN_DIM), jnp.float32),
            jax.ShapeDtypeStruct((n, HIDDEN_DIM), jnp.float32),
        ],
        mesh=mesh,
        scratch_types=[
            pltpu.VMEM((n_chunks, CHUNK), jnp.int32),
            pltpu.VMEM((CHUNK, 2 * HIDDEN_DIM), jnp.float32),
            pltpu.VMEM((CHUNK, HIDDEN_DIM), jnp.float32),
            pltpu.VMEM((CHUNK, HIDDEN_DIM), jnp.float32),
            pltpu.SemaphoreType.DMA,
        ],
    )(_body)

    ax, ar = run(entity_embedding, idx2d)
    return ax.reshape(b, l, HIDDEN_DIM), ar.reshape(b, l, HIDDEN_DIM)


# speedup vs baseline: 3.6956x; 1.3321x over previous
import functools

import jax
import jax.numpy as jnp
from jax import lax
from jax.experimental import pallas as pl
from jax.experimental.pallas import tpu as pltpu
from jax.experimental.pallas import tpu_sc as plsc

PI = 3.141592653589793
GAMMA = 12.0
HIDDEN_DIM = 64
EMBEDDING_RANGE = (GAMMA + 2.0) / HIDDEN_DIM

C_AX = 2.0 * PI / EMBEDDING_RANGE
C_AR = 4.0 * PI / EMBEDDING_RANGE

NW = 32
CB = 8
L = 20
CHUNK = CB * L
IDXW = 80


def _body(table_hbm, idx_hbm, ax_hbm, ar_hbm, idx_v, rows_v, ax_v, ar_v, sem):
    wid = lax.axis_index("s") * 2 + lax.axis_index("c")
    bpw = ax_hbm.shape[0] // NW
    n_chunks = bpw // CB
    idx_rows = idx_v.shape[0]

    pltpu.sync_copy(idx_hbm.at[pl.ds(wid * idx_rows, idx_rows)], idx_v)

    def chunk_step(c, carry):
        cp1 = pltpu.async_copy(
            table_hbm.at[idx_v.at[2 * c]], rows_v.at[pl.ds(0, IDXW)], sem
        )
        cp2 = pltpu.async_copy(
            table_hbm.at[idx_v.at[2 * c + 1]], rows_v.at[pl.ds(IDXW, IDXW)], sem
        )
        cp1.wait()
        cp2.wait()

        @plsc.parallel_loop(0, CHUNK, step=1, unroll=4)
        def row_step(r):
            b = r // L
            t = r - b * L
            for j in range(4):
                x = rows_v[r, pl.ds(j * 16, 16)]
                e = jnp.exp(x * C_AX)
                ax_v[b, t, pl.ds(j * 16, 16)] = PI - (2.0 * PI) / (e + 1.0)
            for j in range(4):
                x = rows_v[r, pl.ds(64 + j * 16, 16)]
                u = jnp.exp(x * C_AR)
                ar_v[b, t, pl.ds(j * 16, 16)] = PI - PI / (u + 1.0)

        base = wid * bpw + c * CB
        pltpu.sync_copy(ax_v, ax_hbm.at[pl.ds(base, CB)])
        pltpu.sync_copy(ar_v, ar_hbm.at[pl.ds(base, CB)])
        return carry

    lax.fori_loop(0, n_chunks, chunk_step, 0)


def kernel(entity_embedding, indices):
    b, l = indices.shape
    n = b * l
    assert l == L and b % (NW * CB) == 0 and n % IDXW == 0
    idx2d = indices.reshape(n // IDXW, IDXW)
    idx_rows_per_w = idx2d.shape[0] // NW

    mesh = plsc.VectorSubcoreMesh(core_axis_name="c", subcore_axis_name="s")
    run = functools.partial(
        pl.kernel,
        out_type=[
            jax.ShapeDtypeStruct((b, L, HIDDEN_DIM), jnp.float32),
            jax.ShapeDtypeStruct((b, L, HIDDEN_DIM), jnp.float32),
        ],
        mesh=mesh,
        scratch_types=[
            pltpu.VMEM((idx_rows_per_w, IDXW), jnp.int32),
            pltpu.VMEM((CHUNK, 2 * HIDDEN_DIM), jnp.float32),
            pltpu.VMEM((CB, L, HIDDEN_DIM), jnp.float32),
            pltpu.VMEM((CB, L, HIDDEN_DIM), jnp.float32),
            pltpu.SemaphoreType.DMA,
        ],
    )(_body)

    return run(entity_embedding, idx2d)
